# Initial kernel scaffold; baseline (speedup 1.0000x reference)
import jax
import jax.numpy as jnp
from jax.experimental import pallas as pl


def kernel(x, edge_index, edge_attr, batch, W_e, W_gnn, b_gnn, W1, b1, W2, b2):
    raise NotImplementedError("write your pallas kernel here")


# speedup vs baseline: 1.5252x; 1.5252x over previous
import functools

import jax
import jax.numpy as jnp
from jax import lax
from jax.experimental import pallas as pl
from jax.experimental.pallas import tpu as pltpu
from jax.experimental.pallas import tpu_sc as plsc

N_NODES = 10000
N_EDGES = 320000
D_FEAT = 128
D_EDGE = 16
N_GRAPHS = 512

NC = 2
NS = 16
EPAD = 327680
EPW = EPAD // NS
CHUNK = 64
NCHUNK = EPW // CHUNK
NPAD = 10240
HALF = NPAD // NC
ACC = HALF + 8
RPS = HALF // NS


def _sc_body(x_hbm, src_hbm, dst_hbm, attr_hbm, zx_hbm, aggx_out, comb_out,
             src_v, dst_v, rows_v, attr_raw, comb_v, aggx_sh, comb_sh, sem):
    c = lax.axis_index("c")
    s = lax.axis_index("s")
    r0 = s * RPS

    pltpu.sync_copy(zx_hbm.at[pl.ds(0, CHUNK)], rows_v)
    z16 = jnp.zeros((16,), jnp.float32)
    o16 = jnp.ones((16,), jnp.float32)

    def crow(k, carry):
        comb_v[k, pl.ds(0, 16)] = z16
        comb_v[k, pl.ds(16, 16)] = o16
        for j in range(2, 8):
            comb_v[k, pl.ds(16 * j, 16)] = z16
        return carry

    lax.fori_loop(0, CHUNK, crow, 0)

    for k in range(RPS // CHUNK):
        rk = r0 + k * CHUNK
        pltpu.sync_copy(rows_v, aggx_sh.at[pl.ds(rk, CHUNK)])
        pltpu.sync_copy(rows_v, comb_sh.at[pl.ds(rk, CHUNK)])

    @pl.when(s == 0)
    def _ztrash():
        pltpu.sync_copy(rows_v, aggx_sh.at[pl.ds(ACC - CHUNK, CHUNK)])
        pltpu.sync_copy(rows_v, comb_sh.at[pl.ds(ACC - CHUNK, CHUNK)])

    plsc.subcore_barrier()

    AW = CHUNK * D_EDGE // 128
    cbase = c * HALF

    def step(j, carry):
        base = pl.multiple_of(s * EPW + j * CHUNK, 8)
        wbase = pl.multiple_of(s * (EPW * D_EDGE // 128) + j * AW, 8)
        pltpu.sync_copy(src_hbm.at[pl.ds(base, CHUNK)], src_v)
        pltpu.sync_copy(dst_hbm.at[pl.ds(base, CHUNK)], dst_v)
        pltpu.async_copy(x_hbm.at[src_v], rows_v, sem).wait()
        pltpu.sync_copy(attr_hbm.at[pl.ds(wbase, AW)], attr_raw)
        for k in range(AW):
            for jj in range(8):
                comb_v[8 * k + jj, pl.ds(0, 16)] = attr_raw[k, pl.ds(16 * jj, 16)]
        for t in range(CHUNK // 16):
            d = dst_v[pl.ds(16 * t, 16)] - cbase
            ok = (d >= 0) & (d < HALF)
            dst_v[pl.ds(16 * t, 16)] = jnp.where(ok, d, HALF)
        pltpu.sync_copy(rows_v, aggx_sh.at[dst_v], add=True)
        pltpu.sync_copy(comb_v, comb_sh.at[dst_v], add=True)
        return carry

    lax.fori_loop(0, NCHUNK, step, 0)
    plsc.subcore_barrier()

    for k in range(RPS // CHUNK):
        rk = r0 + k * CHUNK
        pltpu.sync_copy(aggx_sh.at[pl.ds(rk, CHUNK)], rows_v)
        pltpu.sync_copy(rows_v, aggx_out.at[pl.ds(cbase + rk, CHUNK)])
        pltpu.sync_copy(comb_sh.at[pl.ds(rk, CHUNK)], rows_v)
        pltpu.sync_copy(rows_v, comb_out.at[pl.ds(cbase + rk, CHUNK)])


@jax.jit
def _sc_scatter(x, src, dst, attr_wide, zx):
    mesh = plsc.VectorSubcoreMesh(core_axis_name="c", subcore_axis_name="s",
                                  num_cores=NC, num_subcores=NS)
    f = pl.kernel(
        _sc_body,
        out_type=(
            jax.ShapeDtypeStruct((NPAD, D_FEAT), jnp.float32),
            jax.ShapeDtypeStruct((NPAD, 128), jnp.float32),
        ),
        mesh=mesh,
        scratch_types=[
            pltpu.VMEM((CHUNK,), jnp.int32),
            pltpu.VMEM((CHUNK,), jnp.int32),
            pltpu.VMEM((CHUNK, D_FEAT), jnp.float32),
            pltpu.VMEM((CHUNK * D_EDGE // 128, 128), jnp.float32),
            pltpu.VMEM((CHUNK, 128), jnp.float32),
            pltpu.VMEM_SHARED((ACC, D_FEAT), jnp.float32),
            pltpu.VMEM_SHARED((ACC, 128), jnp.float32),
            pltpu.SemaphoreType.DMA,
        ],
    )
    return f(x, src, dst, attr_wide, zx)


ROWS_B = 2048
NSTEP = NPAD // ROWS_B


def _tc_body(aggx_ref, comb_ref, batch_ref, we_ref, wg_ref, bg_ref,
             w1_ref, b1_ref, w2_ref, b2_ref, z_ref, sums_ref, cnts_ref):
    i = pl.program_id(0)

    @pl.when(i == 0)
    def _init():
        sums_ref[...] = jnp.zeros_like(sums_ref)
        cnts_ref[...] = jnp.zeros_like(cnts_ref)

    aggx = aggx_ref[...]
    agge = comb_ref[:, 0:D_EDGE]
    deg = comb_ref[:, D_EDGE:D_EDGE + 1]
    agg = aggx + jnp.dot(agge, we_ref[...],
                         preferred_element_type=jnp.float32)
    agg = agg / jnp.maximum(deg, 1.0)
    h = jnp.maximum(jnp.dot(agg, wg_ref[...],
                            preferred_element_type=jnp.float32)
                    + bg_ref[...], 0.0)

    gid = lax.broadcasted_iota(jnp.int32, (ROWS_B, N_GRAPHS), 1)
    p = (batch_ref[...] == gid).astype(jnp.float32)
    dn = (((0,), (0,)), ((), ()))
    sums_ref[...] += lax.dot_general(p, h, dn,
                                     preferred_element_type=jnp.float32)
    ones = jnp.ones((ROWS_B, D_FEAT), jnp.float32)
    cnts_ref[...] += lax.dot_general(p, ones, dn,
                                     preferred_element_type=jnp.float32)

    @pl.when(i == NSTEP - 1)
    def _head():
        pooled = sums_ref[...] / jnp.maximum(cnts_ref[...], 1.0)
        z1 = jnp.maximum(jnp.dot(pooled, w1_ref[...],
                                 preferred_element_type=jnp.float32)
                         + b1_ref[...], 0.0)
        z_ref[...] = jnp.dot(z1, w2_ref[...],
                             preferred_element_type=jnp.float32) + b2_ref[...]


@jax.jit
def _tc_head(aggx_p, comb_p, batch2, W_e, W_gnn, b_gnn2, W1, b1_2,
             W2, b2_2):
    grid = (NSTEP,)
    full = lambda shape: pl.BlockSpec(shape, lambda i: (0,) * len(shape))
    return pl.pallas_call(
        _tc_body,
        grid=grid,
        in_specs=[
            pl.BlockSpec((ROWS_B, D_FEAT), lambda i: (i, 0)),
            pl.BlockSpec((ROWS_B, 128), lambda i: (i, 0)),
            pl.BlockSpec((ROWS_B, 1), lambda i: (i, 0)),
            full((D_EDGE, D_FEAT)),
            full((D_FEAT, D_FEAT)),
            full((1, D_FEAT)),
            full((D_FEAT, D_FEAT)),
            full((1, D_FEAT)),
            full((D_FEAT, D_FEAT)),
            full((1, D_FEAT)),
        ],
        out_specs=pl.BlockSpec((N_GRAPHS, D_FEAT), lambda i: (0, 0)),
        out_shape=jax.ShapeDtypeStruct((N_GRAPHS, D_FEAT), jnp.float32),
        scratch_shapes=[
            pltpu.VMEM((N_GRAPHS, D_FEAT), jnp.float32),
            pltpu.VMEM((N_GRAPHS, D_FEAT), jnp.float32),
        ],
    )(aggx_p, comb_p, batch2, W_e, W_gnn, b_gnn2, W1, b1_2, W2, b2_2)


def kernel(x, edge_index, edge_attr, batch, W_e, W_gnn, b_gnn, W1, b1, W2,
           b2):
    x = x.astype(jnp.float32)
    npadedge = EPAD - N_EDGES
    src = jnp.concatenate(
        [edge_index[0].astype(jnp.int32), jnp.zeros((npadedge,), jnp.int32)])
    dst = jnp.concatenate(
        [edge_index[1].astype(jnp.int32),
         jnp.full((npadedge,), NPAD - 1, jnp.int32)])
    attr_wide = jnp.concatenate(
        [edge_attr.astype(jnp.float32),
         jnp.zeros((npadedge, D_EDGE), jnp.float32)]).reshape(
             EPAD * D_EDGE // 128, 128)
    zx = jnp.zeros((NPAD, D_FEAT), jnp.float32)
    aggx_p, comb_p = _sc_scatter(x, src, dst, attr_wide, zx)
    batch_pad = jnp.full((NPAD, 1), N_GRAPHS, jnp.int32)
    batch_pad = batch_pad.at[:N_NODES, 0].set(batch.astype(jnp.int32))
    return _tc_head(aggx_p, comb_p, batch_pad,
                    W_e, W_gnn, b_gnn.reshape(1, D_FEAT),
                    W1, b1.reshape(1, D_FEAT), W2, b2.reshape(1, D_FEAT))


# speedup vs baseline: 1.7915x; 1.1746x over previous
import functools

import jax
import jax.numpy as jnp
from jax import lax
from jax.experimental import pallas as pl
from jax.experimental.pallas import tpu as pltpu
from jax.experimental.pallas import tpu_sc as plsc

N_NODES = 10000
N_EDGES = 320000
D_FEAT = 128
D_EDGE = 16
N_GRAPHS = 512

NC = 2
NS = 16
EPAD = 327680
EPW = EPAD // NS
CHUNK = 64
NCHUNK = EPW // CHUNK
NPAD = 10240
HALF = NPAD // NC
ACC = HALF + 8
RPS = HALF // NS


def _sc_body(x_hbm, src_hbm, dst_hbm, attr_hbm, zx_hbm, aggx_out, comb_out,
             src_v, dst_v, rows_v, attr_raw, comb_v,
             src_b, dst_b, rows_b, attr_b, comb_b,
             aggx_sh, comb_sh, sem, sem_b):
    c = lax.axis_index("c")
    s = lax.axis_index("s")
    r0 = s * RPS

    pltpu.sync_copy(zx_hbm.at[pl.ds(0, CHUNK)], rows_v)
    z16 = jnp.zeros((16,), jnp.float32)
    o16 = jnp.ones((16,), jnp.float32)

    def crow(k, carry):
        for cv in (comb_v, comb_b):
            cv[k, pl.ds(0, 16)] = z16
            cv[k, pl.ds(16, 16)] = o16
            for j in range(2, 8):
                cv[k, pl.ds(16 * j, 16)] = z16
        return carry

    lax.fori_loop(0, CHUNK, crow, 0)

    for k in range(RPS // CHUNK):
        rk = r0 + k * CHUNK
        pltpu.sync_copy(rows_v, aggx_sh.at[pl.ds(rk, CHUNK)])
        pltpu.sync_copy(rows_v, comb_sh.at[pl.ds(rk, CHUNK)])

    @pl.when(s == 0)
    def _ztrash():
        pltpu.sync_copy(rows_v, aggx_sh.at[pl.ds(ACC - CHUNK, CHUNK)])
        pltpu.sync_copy(rows_v, comb_sh.at[pl.ds(ACC - CHUNK, CHUNK)])

    plsc.subcore_barrier()

    AW = CHUNK * D_EDGE // 128
    cbase = c * HALF

    def prep(j, sv, dv, rv, av, cv, sem_g):
        jc = jnp.minimum(j, NCHUNK - 1)
        base = pl.multiple_of(s * EPW + jc * CHUNK, 8)
        wbase = pl.multiple_of(s * (EPW * D_EDGE // 128) + jc * AW, 8)
        pltpu.sync_copy(src_hbm.at[pl.ds(base, CHUNK)], sv)
        pltpu.sync_copy(dst_hbm.at[pl.ds(base, CHUNK)], dv)
        pltpu.async_copy(x_hbm.at[sv], rv, sem_g)
        pltpu.sync_copy(attr_hbm.at[pl.ds(wbase, AW)], av)
        for k in range(AW):
            for jj in range(8):
                cv[8 * k + jj, pl.ds(0, 16)] = av[k, pl.ds(16 * jj, 16)]
        for t in range(CHUNK // 16):
            d = dv[pl.ds(16 * t, 16)] - cbase
            ok = (d >= 0) & (d < HALF)
            dv[pl.ds(16 * t, 16)] = jnp.where(ok, d, HALF)

    def fin(sv, dv, rv, cv, sem_g):
        pltpu.make_async_copy(x_hbm.at[sv], rv, sem_g).wait()
        pltpu.sync_copy(rv, aggx_sh.at[dv], add=True)
        pltpu.sync_copy(cv, comb_sh.at[dv], add=True)

    bufa = (src_v, dst_v, rows_v, attr_raw, comb_v)
    bufb = (src_b, dst_b, rows_b, attr_b, comb_b)
    prep(0, *bufa, sem)

    def outer(jo, carry):
        j0 = 2 * jo
        prep(j0 + 1, *bufb, sem_b)
        fin(src_v, dst_v, rows_v, comb_v, sem)
        prep(j0 + 2, *bufa, sem)
        fin(src_b, dst_b, rows_b, comb_b, sem_b)
        return carry

    lax.fori_loop(0, NCHUNK // 2, outer, 0)
    pltpu.make_async_copy(x_hbm.at[src_v], rows_v, sem).wait()
    plsc.subcore_barrier()

    for k in range(RPS // CHUNK):
        rk = r0 + k * CHUNK
        pltpu.sync_copy(aggx_sh.at[pl.ds(rk, CHUNK)], rows_v)
        pltpu.sync_copy(rows_v, aggx_out.at[pl.ds(cbase + rk, CHUNK)])
        pltpu.sync_copy(comb_sh.at[pl.ds(rk, CHUNK)], rows_v)
        pltpu.sync_copy(rows_v, comb_out.at[pl.ds(cbase + rk, CHUNK)])


@jax.jit
def _sc_scatter(x, src, dst, attr_wide, zx):
    mesh = plsc.VectorSubcoreMesh(core_axis_name="c", subcore_axis_name="s",
                                  num_cores=NC, num_subcores=NS)
    f = pl.kernel(
        _sc_body,
        out_type=(
            jax.ShapeDtypeStruct((NPAD, D_FEAT), jnp.float32),
            jax.ShapeDtypeStruct((NPAD, 128), jnp.float32),
        ),
        mesh=mesh,
        scratch_types=[
            pltpu.VMEM((CHUNK,), jnp.int32),
            pltpu.VMEM((CHUNK,), jnp.int32),
            pltpu.VMEM((CHUNK, D_FEAT), jnp.float32),
            pltpu.VMEM((CHUNK * D_EDGE // 128, 128), jnp.float32),
            pltpu.VMEM((CHUNK, 128), jnp.float32),
            pltpu.VMEM((CHUNK,), jnp.int32),
            pltpu.VMEM((CHUNK,), jnp.int32),
            pltpu.VMEM((CHUNK, D_FEAT), jnp.float32),
            pltpu.VMEM((CHUNK * D_EDGE // 128, 128), jnp.float32),
            pltpu.VMEM((CHUNK, 128), jnp.float32),
            pltpu.VMEM_SHARED((ACC, D_FEAT), jnp.float32),
            pltpu.VMEM_SHARED((ACC, 128), jnp.float32),
            pltpu.SemaphoreType.DMA,
            pltpu.SemaphoreType.DMA,
        ],
    )
    return f(x, src, dst, attr_wide, zx)


ROWS_B = 2048
NSTEP = NPAD // ROWS_B


def _tc_body(aggx_ref, comb_ref, batch_ref, we_ref, wg_ref, bg_ref,
             w1_ref, b1_ref, w2_ref, b2_ref, z_ref, sums_ref, cnts_ref):
    i = pl.program_id(0)

    @pl.when(i == 0)
    def _init():
        sums_ref[...] = jnp.zeros_like(sums_ref)
        cnts_ref[...] = jnp.zeros_like(cnts_ref)

    aggx = aggx_ref[...]
    agge = comb_ref[:, 0:D_EDGE]
    deg = comb_ref[:, D_EDGE:D_EDGE + 1]
    agg = aggx + jnp.dot(agge, we_ref[...],
                         preferred_element_type=jnp.float32)
    agg = agg / jnp.maximum(deg, 1.0)
    h = jnp.maximum(jnp.dot(agg, wg_ref[...],
                            preferred_element_type=jnp.float32)
                    + bg_ref[...], 0.0)

    gid = lax.broadcasted_iota(jnp.int32, (ROWS_B, N_GRAPHS), 1)
    p = (batch_ref[...] == gid).astype(jnp.float32)
    dn = (((0,), (0,)), ((), ()))
    sums_ref[...] += lax.dot_general(p, h, dn,
                                     preferred_element_type=jnp.float32)
    ones = jnp.ones((ROWS_B, D_FEAT), jnp.float32)
    cnts_ref[...] += lax.dot_general(p, ones, dn,
                                     preferred_element_type=jnp.float32)

    @pl.when(i == NSTEP - 1)
    def _head():
        pooled = sums_ref[...] / jnp.maximum(cnts_ref[...], 1.0)
        z1 = jnp.maximum(jnp.dot(pooled, w1_ref[...],
                                 preferred_element_type=jnp.float32)
                         + b1_ref[...], 0.0)
        z_ref[...] = jnp.dot(z1, w2_ref[...],
                             preferred_element_type=jnp.float32) + b2_ref[...]


@jax.jit
def _tc_head(aggx_p, comb_p, batch2, W_e, W_gnn, b_gnn2, W1, b1_2,
             W2, b2_2):
    grid = (NSTEP,)
    full = lambda shape: pl.BlockSpec(shape, lambda i: (0,) * len(shape))
    return pl.pallas_call(
        _tc_body,
        grid=grid,
        in_specs=[
            pl.BlockSpec((ROWS_B, D_FEAT), lambda i: (i, 0)),
            pl.BlockSpec((ROWS_B, 128), lambda i: (i, 0)),
            pl.BlockSpec((ROWS_B, 1), lambda i: (i, 0)),
            full((D_EDGE, D_FEAT)),
            full((D_FEAT, D_FEAT)),
            full((1, D_FEAT)),
            full((D_FEAT, D_FEAT)),
            full((1, D_FEAT)),
            full((D_FEAT, D_FEAT)),
            full((1, D_FEAT)),
        ],
        out_specs=pl.BlockSpec((N_GRAPHS, D_FEAT), lambda i: (0, 0)),
        out_shape=jax.ShapeDtypeStruct((N_GRAPHS, D_FEAT), jnp.float32),
        scratch_shapes=[
            pltpu.VMEM((N_GRAPHS, D_FEAT), jnp.float32),
            pltpu.VMEM((N_GRAPHS, D_FEAT), jnp.float32),
        ],
    )(aggx_p, comb_p, batch2, W_e, W_gnn, b_gnn2, W1, b1_2, W2, b2_2)


def kernel(x, edge_index, edge_attr, batch, W_e, W_gnn, b_gnn, W1, b1, W2,
           b2):
    x = x.astype(jnp.float32)
    npadedge = EPAD - N_EDGES
    src = jnp.concatenate(
        [edge_index[0].astype(jnp.int32), jnp.zeros((npadedge,), jnp.int32)])
    dst = jnp.concatenate(
        [edge_index[1].astype(jnp.int32),
         jnp.full((npadedge,), NPAD - 1, jnp.int32)])
    attr_wide = jnp.concatenate(
        [edge_attr.astype(jnp.float32),
         jnp.zeros((npadedge, D_EDGE), jnp.float32)]).reshape(
             EPAD * D_EDGE // 128, 128)
    zx = jnp.zeros((NPAD, D_FEAT), jnp.float32)
    aggx_p, comb_p = _sc_scatter(x, src, dst, attr_wide, zx)
    batch_pad = jnp.full((NPAD, 1), N_GRAPHS, jnp.int32)
    batch_pad = batch_pad.at[:N_NODES, 0].set(batch.astype(jnp.int32))
    return _tc_head(aggx_p, comb_p, batch_pad,
                    W_e, W_gnn, b_gnn.reshape(1, D_FEAT),
                    W1, b1.reshape(1, D_FEAT), W2, b2.reshape(1, D_FEAT))


# speedup vs baseline: 2.0479x; 1.1431x over previous
import functools

import jax
import jax.numpy as jnp
from jax import lax
from jax.experimental import pallas as pl
from jax.experimental.pallas import tpu as pltpu
from jax.experimental.pallas import tpu_sc as plsc

N_NODES = 10000
N_EDGES = 320000
D_FEAT = 128
D_EDGE = 16
N_GRAPHS = 512

NC = 2
NS = 16
EPAD = 327680
EPW = EPAD // NS
CHUNK = 64
NCHUNK = EPW // CHUNK
NPAD = 10240
HALF = NPAD // NC
ACC = HALF + 8
RPS = HALF // NS


def _sc_body(x_hbm, src_hbm, dst_hbm, attr_hbm, zx_hbm, aggx_out, comb_out,
             src_v, dst_v, rows_v, attr_raw, comb_v,
             src_b, dst_b, rows_b, attr_b, comb_b, dstm_v, dstm_b,
             aggx_sh, comb_sh, sem_ia, sem_ib, sem_ga, sem_gb,
             sem_sa, sem_sb):
    c = lax.axis_index("c")
    s = lax.axis_index("s")
    r0 = s * RPS

    pltpu.sync_copy(zx_hbm.at[pl.ds(0, CHUNK)], rows_v)
    z16 = jnp.zeros((16,), jnp.float32)
    o16 = jnp.ones((16,), jnp.float32)

    def crow(k, carry):
        for cv in (comb_v, comb_b):
            cv[k, pl.ds(0, 16)] = z16
            cv[k, pl.ds(16, 16)] = o16
            for j in range(2, 8):
                cv[k, pl.ds(16 * j, 16)] = z16
        return carry

    lax.fori_loop(0, CHUNK, crow, 0)

    for k in range(RPS // CHUNK):
        rk = r0 + k * CHUNK
        pltpu.sync_copy(rows_v, aggx_sh.at[pl.ds(rk, CHUNK)])
        pltpu.sync_copy(rows_v, comb_sh.at[pl.ds(rk, CHUNK)])

    @pl.when(s == 0)
    def _ztrash():
        pltpu.sync_copy(rows_v, aggx_sh.at[pl.ds(ACC - CHUNK, CHUNK)])
        pltpu.sync_copy(rows_v, comb_sh.at[pl.ds(ACC - CHUNK, CHUNK)])

    plsc.subcore_barrier()

    AW = CHUNK * D_EDGE // 128
    cbase = c * HALF
    EPWW = EPW * D_EDGE // 128

    def io_slices(j):
        jc = jnp.minimum(j, NCHUNK - 1)
        base = pl.multiple_of(s * EPW + jc * CHUNK, 8)
        wbase = pl.multiple_of(s * EPWW + jc * AW, 8)
        return base, wbase

    def prefetch_io(j, sv, dv, av, sem_i):
        base, wbase = io_slices(j)
        pltpu.async_copy(src_hbm.at[pl.ds(base, CHUNK)], sv, sem_i)
        pltpu.async_copy(dst_hbm.at[pl.ds(base, CHUNK)], dv, sem_i)
        pltpu.async_copy(attr_hbm.at[pl.ds(wbase, AW)], av, sem_i)

    def process(j, sv, dv, dmv, rv, av, cv, sem_i, sem_g, sem_s, first):
        base, wbase = io_slices(j)
        pltpu.make_async_copy(src_hbm.at[pl.ds(base, CHUNK)], sv, sem_i).wait()
        pltpu.make_async_copy(dst_hbm.at[pl.ds(base, CHUNK)], dv, sem_i).wait()
        pltpu.make_async_copy(attr_hbm.at[pl.ds(wbase, AW)], av, sem_i).wait()
        if not first:
            pltpu.make_async_copy(rv, aggx_sh.at[dmv], sem_s).wait()
            pltpu.make_async_copy(cv, comb_sh.at[dmv], sem_s).wait()
        pltpu.async_copy(x_hbm.at[sv], rv, sem_g)
        for t in range(CHUNK // 16):
            d = dv[pl.ds(16 * t, 16)] - cbase
            ok = (d >= 0) & (d < HALF)
            dmv[pl.ds(16 * t, 16)] = jnp.where(ok, d, HALF)
        for k in range(AW):
            for jj in range(8):
                cv[8 * k + jj, pl.ds(0, 16)] = av[k, pl.ds(16 * jj, 16)]
        pltpu.make_async_copy(x_hbm.at[sv], rv, sem_g).wait()
        pltpu.async_copy(rv, aggx_sh.at[dmv], sem_s, add=True)
        pltpu.async_copy(cv, comb_sh.at[dmv], sem_s, add=True)
        prefetch_io(j + 2, sv, dv, av, sem_i)

    bufa = (src_v, dst_v, dstm_v, rows_v, attr_raw, comb_v,
            sem_ia, sem_ga, sem_sa)
    bufb = (src_b, dst_b, dstm_b, rows_b, attr_b, comb_b,
            sem_ib, sem_gb, sem_sb)
    prefetch_io(0, src_v, dst_v, attr_raw, sem_ia)
    prefetch_io(1, src_b, dst_b, attr_b, sem_ib)
    process(0, *bufa, True)
    process(1, *bufb, True)

    def outer(jo, carry):
        process(2 * jo, *bufa, False)
        process(2 * jo + 1, *bufb, False)
        return carry

    lax.fori_loop(1, NCHUNK // 2, outer, 0)

    for sv, dv, dmv, rv, av, cv, sem_i, sem_g, sem_s in (bufa, bufb):
        base, wbase = io_slices(NCHUNK)
        pltpu.make_async_copy(rv, aggx_sh.at[dmv], sem_s).wait()
        pltpu.make_async_copy(cv, comb_sh.at[dmv], sem_s).wait()
        pltpu.make_async_copy(src_hbm.at[pl.ds(base, CHUNK)], sv, sem_i).wait()
        pltpu.make_async_copy(dst_hbm.at[pl.ds(base, CHUNK)], dv, sem_i).wait()
        pltpu.make_async_copy(attr_hbm.at[pl.ds(wbase, AW)], av, sem_i).wait()

    plsc.subcore_barrier()

    for k in range(RPS // CHUNK):
        rk = r0 + k * CHUNK
        pltpu.sync_copy(aggx_sh.at[pl.ds(rk, CHUNK)], rows_v)
        pltpu.sync_copy(rows_v, aggx_out.at[pl.ds(cbase + rk, CHUNK)])
        pltpu.sync_copy(comb_sh.at[pl.ds(rk, CHUNK)], rows_v)
        pltpu.sync_copy(rows_v, comb_out.at[pl.ds(cbase + rk, CHUNK)])


@jax.jit
def _sc_scatter(x, src, dst, attr_wide, zx):
    mesh = plsc.VectorSubcoreMesh(core_axis_name="c", subcore_axis_name="s",
                                  num_cores=NC, num_subcores=NS)
    f = pl.kernel(
        _sc_body,
        out_type=(
            jax.ShapeDtypeStruct((NPAD, D_FEAT), jnp.float32),
            jax.ShapeDtypeStruct((NPAD, 128), jnp.float32),
        ),
        mesh=mesh,
        scratch_types=[
            pltpu.VMEM((CHUNK,), jnp.int32),
            pltpu.VMEM((CHUNK,), jnp.int32),
            pltpu.VMEM((CHUNK, D_FEAT), jnp.float32),
            pltpu.VMEM((CHUNK * D_EDGE // 128, 128), jnp.float32),
            pltpu.VMEM((CHUNK, 128), jnp.float32),
            pltpu.VMEM((CHUNK,), jnp.int32),
            pltpu.VMEM((CHUNK,), jnp.int32),
            pltpu.VMEM((CHUNK, D_FEAT), jnp.float32),
            pltpu.VMEM((CHUNK * D_EDGE // 128, 128), jnp.float32),
            pltpu.VMEM((CHUNK, 128), jnp.float32),
            pltpu.VMEM((CHUNK,), jnp.int32),
            pltpu.VMEM((CHUNK,), jnp.int32),
            pltpu.VMEM_SHARED((ACC, D_FEAT), jnp.float32),
            pltpu.VMEM_SHARED((ACC, 128), jnp.float32),
            pltpu.SemaphoreType.DMA,
            pltpu.SemaphoreType.DMA,
            pltpu.SemaphoreType.DMA,
            pltpu.SemaphoreType.DMA,
            pltpu.SemaphoreType.DMA,
            pltpu.SemaphoreType.DMA,
        ],
    )
    return f(x, src, dst, attr_wide, zx)


ROWS_B = 2048
NSTEP = NPAD // ROWS_B


def _tc_body(aggx_ref, comb_ref, batch_ref, we_ref, wg_ref, bg_ref,
             w1_ref, b1_ref, w2_ref, b2_ref, z_ref, sums_ref, cnts_ref):
    i = pl.program_id(0)

    @pl.when(i == 0)
    def _init():
        sums_ref[...] = jnp.zeros_like(sums_ref)
        cnts_ref[...] = jnp.zeros_like(cnts_ref)

    aggx = aggx_ref[...]
    agge = comb_ref[:, 0:D_EDGE]
    deg = comb_ref[:, D_EDGE:D_EDGE + 1]
    agg = aggx + jnp.dot(agge, we_ref[...],
                         preferred_element_type=jnp.float32)
    agg = agg / jnp.maximum(deg, 1.0)
    h = jnp.maximum(jnp.dot(agg, wg_ref[...],
                            preferred_element_type=jnp.float32)
                    + bg_ref[...], 0.0)

    gid = lax.broadcasted_iota(jnp.int32, (ROWS_B, N_GRAPHS), 1)
    p = (batch_ref[...] == gid).astype(jnp.float32)
    dn = (((0,), (0,)), ((), ()))
    sums_ref[...] += lax.dot_general(p, h, dn,
                                     preferred_element_type=jnp.float32)
    ones = jnp.ones((ROWS_B, D_FEAT), jnp.float32)
    cnts_ref[...] += lax.dot_general(p, ones, dn,
                                     preferred_element_type=jnp.float32)

    @pl.when(i == NSTEP - 1)
    def _head():
        pooled = sums_ref[...] / jnp.maximum(cnts_ref[...], 1.0)
        z1 = jnp.maximum(jnp.dot(pooled, w1_ref[...],
                                 preferred_element_type=jnp.float32)
                         + b1_ref[...], 0.0)
        z_ref[...] = jnp.dot(z1, w2_ref[...],
                             preferred_element_type=jnp.float32) + b2_ref[...]


@jax.jit
def _tc_head(aggx_p, comb_p, batch2, W_e, W_gnn, b_gnn2, W1, b1_2,
             W2, b2_2):
    grid = (NSTEP,)
    full = lambda shape: pl.BlockSpec(shape, lambda i: (0,) * len(shape))
    return pl.pallas_call(
        _tc_body,
        grid=grid,
        in_specs=[
            pl.BlockSpec((ROWS_B, D_FEAT), lambda i: (i, 0)),
            pl.BlockSpec((ROWS_B, 128), lambda i: (i, 0)),
            pl.BlockSpec((ROWS_B, 1), lambda i: (i, 0)),
            full((D_EDGE, D_FEAT)),
            full((D_FEAT, D_FEAT)),
            full((1, D_FEAT)),
            full((D_FEAT, D_FEAT)),
            full((1, D_FEAT)),
            full((D_FEAT, D_FEAT)),
            full((1, D_FEAT)),
        ],
        out_specs=pl.BlockSpec((N_GRAPHS, D_FEAT), lambda i: (0, 0)),
        out_shape=jax.ShapeDtypeStruct((N_GRAPHS, D_FEAT), jnp.float32),
        scratch_shapes=[
            pltpu.VMEM((N_GRAPHS, D_FEAT), jnp.float32),
            pltpu.VMEM((N_GRAPHS, D_FEAT), jnp.float32),
        ],
    )(aggx_p, comb_p, batch2, W_e, W_gnn, b_gnn2, W1, b1_2, W2, b2_2)


def kernel(x, edge_index, edge_attr, batch, W_e, W_gnn, b_gnn, W1, b1, W2,
           b2):
    x = x.astype(jnp.float32)
    npadedge = EPAD - N_EDGES
    src = jnp.concatenate(
        [edge_index[0].astype(jnp.int32), jnp.zeros((npadedge,), jnp.int32)])
    dst = jnp.concatenate(
        [edge_index[1].astype(jnp.int32),
         jnp.full((npadedge,), NPAD - 1, jnp.int32)])
    attr_wide = jnp.concatenate(
        [edge_attr.astype(jnp.float32),
         jnp.zeros((npadedge, D_EDGE), jnp.float32)]).reshape(
             EPAD * D_EDGE // 128, 128)
    zx = jnp.zeros((NPAD, D_FEAT), jnp.float32)
    aggx_p, comb_p = _sc_scatter(x, src, dst, attr_wide, zx)
    batch_pad = jnp.full((NPAD, 1), N_GRAPHS, jnp.int32)
    batch_pad = batch_pad.at[:N_NODES, 0].set(batch.astype(jnp.int32))
    return _tc_head(aggx_p, comb_p, batch_pad,
                    W_e, W_gnn, b_gnn.reshape(1, D_FEAT),
                    W1, b1.reshape(1, D_FEAT), W2, b2.reshape(1, D_FEAT))
